# baseline (speedup 1.0000x reference)
import jax
import jax.numpy as jnp
from jax.experimental import pallas as pl
from jax.experimental.pallas import tpu as pltpu

_REP = 128


_CHUNK = 16
_NSEM = 8


def _bcast_add_kernel(qpw_ref, q_ref, out_ref, rep_ref, sems):
    s = q_ref[0] + qpw_ref[...]
    rep_ref[...] = jnp.broadcast_to(s[None], rep_ref.shape)
    bs = out_ref.shape[0]
    n = bs // _CHUNK
    nsrc = _REP // _CHUNK
    copies = [
        pltpu.make_async_copy(
            rep_ref.at[pl.ds((i % nsrc) * _CHUNK, _CHUNK)],
            out_ref.at[pl.ds(i * _CHUNK, _CHUNK)],
            sems.at[i % _NSEM],
        )
        for i in range(n)
    ]
    for c in copies:
        c.start()
    for c in copies:
        c.wait()


def kernel(x, query_pos_weight, queries):
    bs = x.shape[0]
    n_query, embed_dim = query_pos_weight.shape
    out = pl.pallas_call(
        _bcast_add_kernel,
        in_specs=[
            pl.BlockSpec(memory_space=pltpu.VMEM),
            pl.BlockSpec(memory_space=pltpu.VMEM),
        ],
        out_specs=pl.BlockSpec(memory_space=pl.ANY),
        out_shape=jax.ShapeDtypeStruct((bs, n_query, embed_dim), queries.dtype),
        scratch_shapes=[
            pltpu.VMEM((_REP, n_query, embed_dim), queries.dtype),
            pltpu.SemaphoreType.DMA((_NSEM,)),
        ],
    )(query_pos_weight, queries)
    return out


# speedup vs baseline: 2.8366x; 2.8366x over previous
import jax
import jax.numpy as jnp
from jax.experimental import pallas as pl

_B_BLK = 128


def _bcast_add_kernel(qpw_ref, q_ref, out_ref):
    s = q_ref[0] + qpw_ref[...]
    out_ref[...] = jnp.broadcast_to(s[:, None, :], out_ref.shape)


def kernel(x, query_pos_weight, queries):
    bs = x.shape[0]
    n_query, embed_dim = query_pos_weight.shape
    grid = (bs // _B_BLK,)
    out = pl.pallas_call(
        _bcast_add_kernel,
        grid=grid,
        in_specs=[
            pl.BlockSpec((n_query, embed_dim), lambda i: (0, 0)),
            pl.BlockSpec((1, n_query, embed_dim), lambda i: (0, 0, 0)),
        ],
        out_specs=pl.BlockSpec((n_query, _B_BLK, embed_dim), lambda i: (0, i, 0)),
        out_shape=jax.ShapeDtypeStruct((n_query, bs, embed_dim), queries.dtype),
    )(query_pos_weight, queries)
    return jnp.swapaxes(out, 0, 1)


# speedup vs baseline: 2.9757x; 1.0490x over previous
import jax
import jax.numpy as jnp
from jax.experimental import pallas as pl

_B_BLK = 64


def _bcast_add_kernel(qpw_ref, q_ref, out_ref):
    s = q_ref[0] + qpw_ref[...]
    out_ref[...] = jnp.broadcast_to(s[:, None, :], out_ref.shape)


def kernel(x, query_pos_weight, queries):
    bs = x.shape[0]
    n_query, embed_dim = query_pos_weight.shape
    grid = (bs // _B_BLK,)
    out = pl.pallas_call(
        _bcast_add_kernel,
        grid=grid,
        in_specs=[
            pl.BlockSpec((n_query, embed_dim), lambda i: (0, 0)),
            pl.BlockSpec((1, n_query, embed_dim), lambda i: (0, 0, 0)),
        ],
        out_specs=pl.BlockSpec((n_query, _B_BLK, embed_dim), lambda i: (0, i, 0)),
        out_shape=jax.ShapeDtypeStruct((n_query, bs, embed_dim), queries.dtype),
    )(query_pos_weight, queries)
    return jnp.swapaxes(out, 0, 1)
